# baseline (speedup 1.0000x reference)
import functools

import numpy as np
import jax
import jax.numpy as jnp
from jax import lax
from jax.experimental import pallas as pl
from jax.experimental.pallas import tpu as pltpu
from jax.experimental.pallas import tpu_sc as plsc

N = 10000
E = 320000
D = 128
H = 128
T = 2

NC = 2
NS = 16
NW = NC * NS
EW = E // NW
KE = 40
KP = 48
NCH = EW // KE
RPT = 624

BN = 1000
GRID = N // BN


def _bf(v):
    return v.astype(jnp.bfloat16)


def _l1_body(x_ref, w_ref, asrc_ref, adst_ref, h_ref, sd_ref):
    h = jnp.dot(_bf(x_ref[...]), _bf(w_ref[...]),
                preferred_element_type=jnp.float32)
    h_ref[...] = h
    hb = _bf(h).astype(jnp.float32)
    sd_ref[0, 0, :] = jnp.sum(hb * _bf(asrc_ref[...]).astype(jnp.float32), axis=1)
    sd_ref[0, 1, :] = jnp.sum(hb * _bf(adst_ref[...]).astype(jnp.float32), axis=1)


_l1 = pl.pallas_call(
    _l1_body,
    grid=(GRID,),
    in_specs=[
        pl.BlockSpec((BN, D), lambda i: (i, 0)),
        pl.BlockSpec((D, H), lambda i: (0, 0)),
        pl.BlockSpec((1, H), lambda i: (0, 0)),
        pl.BlockSpec((1, H), lambda i: (0, 0)),
    ],
    out_specs=[
        pl.BlockSpec((BN, H), lambda i: (i, 0)),
        pl.BlockSpec((1, 2, BN), lambda i: (i, 0, 0)),
    ],
    out_shape=[
        jax.ShapeDtypeStruct((N, H), jnp.float32),
        jax.ShapeDtypeStruct((GRID, 2, BN), jnp.float32),
    ],
)


def _l2_body(acc_ref, den_ref, b_ref, w_ref, asrc_ref, adst_ref, h_ref, sd_ref):
    den = jnp.sum(den_ref[0], axis=0)
    hs = acc_ref[0] + acc_ref[1]
    g = jnp.maximum(hs / (den[:, None] + 1e-16) + b_ref[...], 0.0)
    h = jnp.dot(_bf(g), _bf(w_ref[...]), preferred_element_type=jnp.float32)
    h_ref[...] = h
    hb = _bf(h).astype(jnp.float32)
    sd_ref[0, 0, :] = jnp.sum(hb * _bf(asrc_ref[...]).astype(jnp.float32), axis=1)
    sd_ref[0, 1, :] = jnp.sum(hb * _bf(adst_ref[...]).astype(jnp.float32), axis=1)


_l2 = pl.pallas_call(
    _l2_body,
    grid=(GRID,),
    in_specs=[
        pl.BlockSpec((2, BN, H), lambda i: (0, i, 0)),
        pl.BlockSpec((1, NW, BN), lambda i: (i, 0, 0)),
        pl.BlockSpec((1, H), lambda i: (0, 0)),
        pl.BlockSpec((H, H), lambda i: (0, 0)),
        pl.BlockSpec((1, H), lambda i: (0, 0)),
        pl.BlockSpec((1, H), lambda i: (0, 0)),
    ],
    out_specs=[
        pl.BlockSpec((BN, H), lambda i: (i, 0)),
        pl.BlockSpec((1, 2, BN), lambda i: (i, 0, 0)),
    ],
    out_shape=[
        jax.ShapeDtypeStruct((N, H), jnp.float32),
        jax.ShapeDtypeStruct((GRID, 2, BN), jnp.float32),
    ],
)


def _l3_body(acc_ref, den_ref, b_ref, wl_ref, bl_ref, o_ref):
    den = jnp.sum(den_ref[0], axis=0)
    hs = acc_ref[0] + acc_ref[1]
    g = hs / (den[:, None] + 1e-16) + b_ref[...]
    o = jnp.dot(_bf(g), _bf(wl_ref[...]),
                preferred_element_type=jnp.float32) + bl_ref[...]
    o_ref[...] = jnp.maximum(o, 0.0)


_l3 = pl.pallas_call(
    _l3_body,
    grid=(GRID,),
    in_specs=[
        pl.BlockSpec((2, BN, H), lambda i: (0, i, 0)),
        pl.BlockSpec((1, NW, BN), lambda i: (i, 0, 0)),
        pl.BlockSpec((1, H), lambda i: (0, 0)),
        pl.BlockSpec((H, T), lambda i: (0, 0)),
        pl.BlockSpec((1, T), lambda i: (0, 0)),
    ],
    out_specs=pl.BlockSpec((BN, T), lambda i: (i, 0)),
    out_shape=jax.ShapeDtypeStruct((N, T), jnp.float32),
)


def _edge_body(h_hbm, asrc_hbm, adst_hbm, pe_hbm,
               acc_hbm, den_hbm,
               asrc_v, adst_v, den_v, idxp_v, srck_v, dstk_v, exk_v, rows_v,
               acc_s, sem_i, sem_g, sem_s):
    cid = lax.axis_index("c")
    sid = lax.axis_index("s")
    wid = sid * NC + cid
    ebase = wid * EW

    zf = jnp.zeros((16,), jnp.float32)
    zi = jnp.zeros((16,), jnp.int32)

    @pl.loop(0, KP)
    def _zr(r):
        for v in range(H // 16):
            rows_v[0, r, pl.ds(v * 16, 16)] = zf

    @pl.loop(0, N // 16)
    def _zd(i):
        den_v[pl.ds(i * 16, 16)] = zf

    for s in range(3):
        idxp_v[s, pl.ds(KP - 16, 16)] = zi
    for b in range(2):
        srck_v[b, pl.ds(KP - 16, 16)] = zi
        dstk_v[b, pl.ds(KP - 16, 16)] = zi

    for t in range(RPT // KP):
        pltpu.sync_copy(rows_v.at[0], acc_s.at[pl.ds(sid * RPT + t * KP, KP)])

    @pl.when(sid == NS - 1)
    def _ztail():
        pltpu.sync_copy(rows_v.at[0, pl.ds(0, 16)], acc_s.at[pl.ds(N - 16, 16)])

    pltpu.sync_copy(asrc_hbm, asrc_v)
    pltpu.sync_copy(adst_hbm, adst_v)

    plsc.subcore_barrier()

    lane = lax.iota(jnp.int32, 16)
    one = jnp.full((16,), 1, jnp.int32)
    onehot = [
        (one - jnp.minimum(jnp.abs(lane - j), one)).astype(jnp.float32)
        for j in range(16)
    ]

    def _exp(e):
        y = jnp.maximum(jnp.minimum(e, 80.0), -80.0) * jnp.float32(
            1.4426950408889634)
        nf = (y + jnp.float32(12582912.0)) - jnp.float32(12582912.0)
        f = y - nf
        p = jnp.float32(0.00015403530393381606)
        p = p * f + jnp.float32(0.0013333558146428443)
        p = p * f + jnp.float32(0.009618129107628477)
        p = p * f + jnp.float32(0.05550410866482158)
        p = p * f + jnp.float32(0.2402265069591007)
        p = p * f + jnp.float32(0.6931471805599453)
        p = p * f + jnp.float32(1.0)
        ni = nf.astype(jnp.int32)
        scale = plsc.bitcast(lax.shift_left(ni + 127, 23), jnp.float32)
        return p * scale

    def _stage(c):
        pltpu.async_copy(pe_hbm.at[pl.ds(ebase + c * KE, KE)],
                         idxp_v.at[lax.rem(c, 3), pl.ds(0, KE)], sem_i)

    def _wait_bytes(dst, sem):
        pltpu.make_async_copy(h_hbm.at[pl.ds(0, dst.shape[0])], dst,
                              sem).wait()

    def _wait_idx():
        pltpu.make_async_copy(pe_hbm.at[pl.ds(0, KE)],
                              idxp_v.at[0, pl.ds(0, KE)], sem_i).wait()

    def _unpack(c, slot):
        s3 = lax.rem(c, 3)
        for g in range(KP // 16):
            p16 = idxp_v[s3, pl.ds(g * 16, 16)]
            srck_v[slot, pl.ds(g * 16, 16)] = lax.shift_right_logical(p16, 14)
            dstk_v[slot, pl.ds(g * 16, 16)] = lax.bitwise_and(
                p16, jnp.int32(16383))

    def _chunk(c, b):
        @pl.when(c + 2 < NCH)
        def _pf():
            _stage(c + 2)

        @pl.when(c >= 1)
        def _ws():
            _wait_bytes(rows_v.at[1 - b], sem_s)

        @pl.when(c + 1 < NCH)
        def _nx():
            _wait_idx()
            _unpack(c + 1, 1 - b)
            pltpu.async_copy(h_hbm.at[srck_v.at[1 - b]], rows_v.at[1 - b],
                             sem_g)

        _wait_bytes(rows_v.at[b], sem_g)

        for g in range(KP // 16):
            si = srck_v[b, pl.ds(g * 16, 16)]
            di = dstk_v[b, pl.ds(g * 16, 16)]
            av = plsc.load_gather(asrc_v, [si])
            bv = plsc.load_gather(adst_v, [di])
            e = av + bv
            e = jnp.where(e >= 0.0, e, e * jnp.float32(0.2))
            ex = _exp(e)
            if (g + 1) * 16 > KE:
                ex = jnp.where(lane < KE - g * 16, ex, 0.0)
            plsc.addupdate_scatter(den_v, [di], ex)
            for j in range(16):
                s = jnp.sum(ex * onehot[j])
                r = g * 16 + j
                for v in range(H // 16):
                    rows_v[b, r, pl.ds(v * 16, 16)] = (
                        rows_v[b, r, pl.ds(v * 16, 16)] * s)

        pltpu.async_copy(rows_v.at[b], acc_s.at[dstk_v.at[b]], sem_s,
                         add=True)

    _stage(0)
    _stage(1)
    _wait_idx()
    _unpack(0, 0)
    pltpu.async_copy(h_hbm.at[srck_v.at[0]], rows_v.at[0], sem_g)

    @pl.loop(0, NCH)
    def _p2(c):
        for b in range(2):
            @pl.when(lax.rem(c, 2) == b)
            def _sel():
                _chunk(c, b)

    _wait_bytes(rows_v.at[(NCH - 1) % 2], sem_s)

    plsc.subcore_barrier()

    pltpu.sync_copy(acc_s.at[pl.ds(sid * RPT, RPT)],
                    acc_hbm.at[cid, pl.ds(sid * RPT, RPT)])

    @pl.when(sid == NS - 1)
    def _ctail():
        pltpu.sync_copy(acc_s.at[pl.ds(N - 16, 16)],
                        acc_hbm.at[cid, pl.ds(N - 16, 16)])

    for g in range(GRID):
        pltpu.sync_copy(den_v.at[pl.ds(g * BN, BN)],
                        den_hbm.at[pl.ds((g * NW + wid) * BN, BN)])


_edge = functools.partial(
    pl.kernel,
    out_type=(jax.ShapeDtypeStruct((NC, N, H), jnp.float32),
              jax.ShapeDtypeStruct((NW * N,), jnp.float32)),
    mesh=plsc.VectorSubcoreMesh(core_axis_name="c", subcore_axis_name="s"),
    scratch_types=(
        pltpu.VMEM((N,), jnp.float32),
        pltpu.VMEM((N,), jnp.float32),
        pltpu.VMEM((N,), jnp.float32),
        pltpu.VMEM((3, KP), jnp.int32),
        pltpu.VMEM((2, KP), jnp.int32),
        pltpu.VMEM((2, KP), jnp.int32),
        pltpu.VMEM((KP,), jnp.float32),
        pltpu.VMEM((2, KP, H), jnp.float32),
        pltpu.VMEM_SHARED((N, H), jnp.float32),
        pltpu.SemaphoreType.DMA,
        pltpu.SemaphoreType.DMA,
        pltpu.SemaphoreType.DMA,
    ),
    compiler_params=pltpu.CompilerParams(needs_layout_passes=False),
)(_edge_body)


def kernel(x, edge_index, W1, as1, ad1, b1, W2, as2, ad2, b2, Wl, bl):
    pe = lax.bitwise_or(lax.shift_left(edge_index[0], 14), edge_index[1])

    h1, sd1 = _l1(x, W1, as1.reshape(1, H), ad1.reshape(1, H))
    acc1, den1 = _edge(h1, sd1[:, 0, :].reshape(N), sd1[:, 1, :].reshape(N),
                       pe)
    h2, sd2 = _l2(acc1, den1.reshape(GRID, NW, BN), b1.reshape(1, H), W2,
                  as2.reshape(1, H), ad2.reshape(1, H))
    acc2, den2 = _edge(h2, sd2[:, 0, :].reshape(N), sd2[:, 1, :].reshape(N),
                       pe)
    out = _l3(acc2, den2.reshape(GRID, NW, BN), b2.reshape(1, H), Wl,
              bl.reshape(1, T))
    return out


# speedup vs baseline: 7.8065x; 7.8065x over previous
import functools

import numpy as np
import jax
import jax.numpy as jnp
from jax import lax
from jax.experimental import pallas as pl
from jax.experimental.pallas import tpu as pltpu
from jax.experimental.pallas import tpu_sc as plsc

N = 10000
E = 320000
D = 128
H = 128
T = 2

NC = 2
NS = 16
NW = NC * NS
EW = E // NW
KE = 80
KP = 80
NCH = EW // KE
RPT = 624

BN = 1000
GRID = N // BN


def _bf(v):
    return v.astype(jnp.bfloat16)


def _l1_body(x_ref, w_ref, asrc_ref, adst_ref, h_ref, sd_ref):
    h = jnp.dot(_bf(x_ref[...]), _bf(w_ref[...]),
                preferred_element_type=jnp.float32)
    h_ref[...] = h
    hb = _bf(h).astype(jnp.float32)
    sd_ref[0, 0, :] = jnp.sum(hb * _bf(asrc_ref[...]).astype(jnp.float32), axis=1)
    sd_ref[0, 1, :] = jnp.sum(hb * _bf(adst_ref[...]).astype(jnp.float32), axis=1)


_l1 = pl.pallas_call(
    _l1_body,
    grid=(GRID,),
    in_specs=[
        pl.BlockSpec((BN, D), lambda i: (i, 0)),
        pl.BlockSpec((D, H), lambda i: (0, 0)),
        pl.BlockSpec((1, H), lambda i: (0, 0)),
        pl.BlockSpec((1, H), lambda i: (0, 0)),
    ],
    out_specs=[
        pl.BlockSpec((BN, H), lambda i: (i, 0)),
        pl.BlockSpec((1, 2, BN), lambda i: (i, 0, 0)),
    ],
    out_shape=[
        jax.ShapeDtypeStruct((N, H), jnp.float32),
        jax.ShapeDtypeStruct((GRID, 2, BN), jnp.float32),
    ],
)


def _l2_body(acc_ref, den_ref, b_ref, w_ref, asrc_ref, adst_ref, h_ref, sd_ref):
    den = jnp.sum(den_ref[0], axis=0)
    hs = acc_ref[0] + acc_ref[1]
    g = jnp.maximum(hs / (den[:, None] + 1e-16) + b_ref[...], 0.0)
    h = jnp.dot(_bf(g), _bf(w_ref[...]), preferred_element_type=jnp.float32)
    h_ref[...] = h
    hb = _bf(h).astype(jnp.float32)
    sd_ref[0, 0, :] = jnp.sum(hb * _bf(asrc_ref[...]).astype(jnp.float32), axis=1)
    sd_ref[0, 1, :] = jnp.sum(hb * _bf(adst_ref[...]).astype(jnp.float32), axis=1)


_l2 = pl.pallas_call(
    _l2_body,
    grid=(GRID,),
    in_specs=[
        pl.BlockSpec((2, BN, H), lambda i: (0, i, 0)),
        pl.BlockSpec((1, NW, BN), lambda i: (i, 0, 0)),
        pl.BlockSpec((1, H), lambda i: (0, 0)),
        pl.BlockSpec((H, H), lambda i: (0, 0)),
        pl.BlockSpec((1, H), lambda i: (0, 0)),
        pl.BlockSpec((1, H), lambda i: (0, 0)),
    ],
    out_specs=[
        pl.BlockSpec((BN, H), lambda i: (i, 0)),
        pl.BlockSpec((1, 2, BN), lambda i: (i, 0, 0)),
    ],
    out_shape=[
        jax.ShapeDtypeStruct((N, H), jnp.float32),
        jax.ShapeDtypeStruct((GRID, 2, BN), jnp.float32),
    ],
)


def _l3_body(acc_ref, den_ref, b_ref, wl_ref, bl_ref, o_ref):
    den = jnp.sum(den_ref[0], axis=0)
    hs = acc_ref[0] + acc_ref[1]
    g = hs / (den[:, None] + 1e-16) + b_ref[...]
    o = jnp.dot(_bf(g), _bf(wl_ref[...]),
                preferred_element_type=jnp.float32) + bl_ref[...]
    o_ref[...] = jnp.maximum(o, 0.0)


_l3 = pl.pallas_call(
    _l3_body,
    grid=(GRID,),
    in_specs=[
        pl.BlockSpec((2, BN, H), lambda i: (0, i, 0)),
        pl.BlockSpec((1, NW, BN), lambda i: (i, 0, 0)),
        pl.BlockSpec((1, H), lambda i: (0, 0)),
        pl.BlockSpec((H, T), lambda i: (0, 0)),
        pl.BlockSpec((1, T), lambda i: (0, 0)),
    ],
    out_specs=pl.BlockSpec((BN, T), lambda i: (i, 0)),
    out_shape=jax.ShapeDtypeStruct((N, T), jnp.float32),
)


def _edge_body(h_hbm, asrc_hbm, adst_hbm, pe_hbm,
               acc_hbm, den_hbm,
               asrc_v, adst_v, den_v, idxp_v, srck_v, dstk_v, rows_v,
               acc_s, sem_i):
    cid = lax.axis_index("c")
    sid = lax.axis_index("s")
    wid = sid * NC + cid
    ebase = wid * EW

    zf = jnp.zeros((16,), jnp.float32)
    zi = jnp.zeros((16,), jnp.int32)

    @pl.loop(0, KP)
    def _zr(r):
        for v in range(H // 16):
            rows_v[r, pl.ds(v * 16, 16)] = zf

    @pl.loop(0, N // 16)
    def _zd(i):
        den_v[pl.ds(i * 16, 16)] = zf

    for t in range(7):
        pltpu.sync_copy(rows_v, acc_s.at[pl.ds(sid * RPT + t * KP, KP)])
    pltpu.sync_copy(rows_v.at[pl.ds(0, 64)],
                    acc_s.at[pl.ds(sid * RPT + 7 * KP, 64)])

    @pl.when(sid == NS - 1)
    def _ztail():
        pltpu.sync_copy(rows_v.at[pl.ds(0, 16)], acc_s.at[pl.ds(N - 16, 16)])

    pltpu.sync_copy(asrc_hbm, asrc_v)
    pltpu.sync_copy(adst_hbm, adst_v)

    plsc.subcore_barrier()

    lane = lax.iota(jnp.int32, 16)
    one = jnp.full((16,), 1, jnp.int32)
    onehot = [
        (one - jnp.minimum(jnp.abs(lane - j), one)).astype(jnp.float32)
        for j in range(16)
    ]

    def _exp(e):
        y = jnp.maximum(jnp.minimum(e, 80.0), -80.0) * jnp.float32(
            1.4426950408889634)
        nf = (y + jnp.float32(12582912.0)) - jnp.float32(12582912.0)
        f = y - nf
        p = jnp.float32(0.00015403530393381606)
        p = p * f + jnp.float32(0.0013333558146428443)
        p = p * f + jnp.float32(0.009618129107628477)
        p = p * f + jnp.float32(0.05550410866482158)
        p = p * f + jnp.float32(0.2402265069591007)
        p = p * f + jnp.float32(0.6931471805599453)
        p = p * f + jnp.float32(1.0)
        ni = nf.astype(jnp.int32)
        scale = plsc.bitcast(lax.shift_left(ni + 127, 23), jnp.float32)
        return p * scale

    def _stage(c):
        pltpu.async_copy(pe_hbm.at[pl.ds(ebase + c * KE, KE)],
                         idxp_v.at[lax.rem(c, 3)], sem_i)

    def _wait_idx():
        pltpu.make_async_copy(pe_hbm.at[pl.ds(0, KE)],
                              idxp_v.at[0], sem_i).wait()

    def _unpack(c):
        s3 = lax.rem(c, 3)
        for g in range(KP // 16):
            p16 = idxp_v[s3, pl.ds(g * 16, 16)]
            srck_v[0, pl.ds(g * 16, 16)] = lax.shift_right_logical(p16, 14)
            dstk_v[0, pl.ds(g * 16, 16)] = lax.bitwise_and(
                p16, jnp.int32(16383))

    _stage(0)
    _stage(1)

    @pl.loop(0, NCH)
    def _p2(c):
        @pl.when(c + 2 < NCH)
        def _pf():
            _stage(c + 2)

        _wait_idx()
        _unpack(c)
        pltpu.sync_copy(h_hbm.at[srck_v.at[0]], rows_v)

        for g in range(KP // 16):
            si = srck_v[0, pl.ds(g * 16, 16)]
            di = dstk_v[0, pl.ds(g * 16, 16)]
            av = plsc.load_gather(asrc_v, [si])
            bv = plsc.load_gather(adst_v, [di])
            e = av + bv
            e = jnp.where(e >= 0.0, e, e * jnp.float32(0.2))
            ex = _exp(e)
            plsc.addupdate_scatter(den_v, [di], ex)
            for j in range(16):
                s = jnp.sum(ex * onehot[j])
                r = g * 16 + j
                for v in range(H // 16):
                    rows_v[r, pl.ds(v * 16, 16)] = (
                        rows_v[r, pl.ds(v * 16, 16)] * s)

        pltpu.sync_copy(rows_v, acc_s.at[dstk_v.at[0]], add=True)

    plsc.subcore_barrier()

    pltpu.sync_copy(acc_s.at[pl.ds(sid * RPT, RPT)],
                    acc_hbm.at[cid, pl.ds(sid * RPT, RPT)])

    @pl.when(sid == NS - 1)
    def _ctail():
        pltpu.sync_copy(acc_s.at[pl.ds(N - 16, 16)],
                        acc_hbm.at[cid, pl.ds(N - 16, 16)])

    for g in range(GRID):
        pltpu.sync_copy(den_v.at[pl.ds(g * BN, BN)],
                        den_hbm.at[pl.ds((g * NW + wid) * BN, BN)])


_edge = functools.partial(
    pl.kernel,
    out_type=(jax.ShapeDtypeStruct((NC, N, H), jnp.float32),
              jax.ShapeDtypeStruct((NW * N,), jnp.float32)),
    mesh=plsc.VectorSubcoreMesh(core_axis_name="c", subcore_axis_name="s"),
    scratch_types=(
        pltpu.VMEM((N,), jnp.float32),
        pltpu.VMEM((N,), jnp.float32),
        pltpu.VMEM((N,), jnp.float32),
        pltpu.VMEM((3, KP), jnp.int32),
        pltpu.VMEM((1, KP), jnp.int32),
        pltpu.VMEM((1, KP), jnp.int32),
        pltpu.VMEM((KP, H), jnp.float32),
        pltpu.VMEM_SHARED((N, H), jnp.float32),
        pltpu.SemaphoreType.DMA,
    ),
    compiler_params=pltpu.CompilerParams(needs_layout_passes=False),
)(_edge_body)


def kernel(x, edge_index, W1, as1, ad1, b1, W2, as2, ad2, b2, Wl, bl):
    pe = lax.bitwise_or(lax.shift_left(edge_index[0], 14), edge_index[1])

    h1, sd1 = _l1(x, W1, as1.reshape(1, H), ad1.reshape(1, H))
    acc1, den1 = _edge(h1, sd1[:, 0, :].reshape(N), sd1[:, 1, :].reshape(N),
                       pe)
    h2, sd2 = _l2(acc1, den1.reshape(GRID, NW, BN), b1.reshape(1, H), W2,
                  as2.reshape(1, H), ad2.reshape(1, H))
    acc2, den2 = _edge(h2, sd2[:, 0, :].reshape(N), sd2[:, 1, :].reshape(N),
                       pe)
    out = _l3(acc2, den2.reshape(GRID, NW, BN), b2.reshape(1, H), Wl,
              bl.reshape(1, T))
    return out
